# baseline (speedup 1.0000x reference)
import jax
import jax.numpy as jnp
from jax import lax
from jax.experimental import pallas as pl
from jax.experimental.pallas import tpu as pltpu

_B, _S, _H = 4, 4096, 2048
_N = _B * _S
_CHR = 128
_STEPS = _N // _CHR
_NBUF = 8


def _body(alpha_ref, h_hbm, m_ref, scale_ref, shift_ref, out_hbm,
          bufs, ld_sem, st_sem):
    def _ld(c, b):
        pltpu.make_async_copy(
            h_hbm.at[pl.ds(c * _CHR, _CHR), :], bufs[b], ld_sem.at[b]).start()

    def _st(c, b):
        pltpu.make_async_copy(
            bufs[b], out_hbm.at[pl.ds(c * _CHR, _CHR), :], st_sem.at[b]).start()

    def _wait_ld(b):
        pltpu.make_async_copy(
            h_hbm.at[pl.ds(0, _CHR), :], bufs[b], ld_sem.at[b]).wait()

    def _wait_st(b):
        pltpu.make_async_copy(
            bufs[b], out_hbm.at[pl.ds(0, _CHR), :], st_sem.at[b]).wait()

    for i in range(_NBUF - 1):
        _ld(i, i)

    alpha = alpha_ref[0]
    scale_row = scale_ref[...]
    shift_row = shift_ref[...]

    def _outer(k2, carry):
        c0 = k2 * _NBUF
        for b in range(_NBUF):
            c = c0 + b
            _wait_ld(b)
            h = bufs[b][...]
            t = jnp.where(m_ref[pl.ds(c * _CHR, _CHR), :] == 2,
                          alpha, jnp.float32(0.0))
            bufs[b][...] = h + t * (h * scale_row + shift_row)
            _st(c, b)

            @pl.when(c + _NBUF - 1 < _STEPS)
            def _prefetch(c=c, b=b):
                @pl.when(c >= 1)
                def _drain():
                    _wait_st((b - 1) % _NBUF)
                _ld(c + _NBUF - 1, (b - 1) % _NBUF)
        return carry

    lax.fori_loop(0, _STEPS // _NBUF, _outer, 0, unroll=False)
    for b in range(_NBUF):
        _wait_st(b)


def kernel(hidden_states, input_mask, scale, shift, alpha):
    h = hidden_states.reshape(_N, _H)
    m = input_mask.reshape(_N, 1)
    scale2 = scale.reshape(1, _H)
    shift2 = shift.reshape(1, _H)
    alpha1 = jnp.asarray(alpha, jnp.float32).reshape(1)

    out = pl.pallas_call(
        _body,
        in_specs=[
            pl.BlockSpec(memory_space=pltpu.SMEM),
            pl.BlockSpec(memory_space=pl.ANY),
            pl.BlockSpec(memory_space=pltpu.VMEM),
            pl.BlockSpec(memory_space=pltpu.VMEM),
            pl.BlockSpec(memory_space=pltpu.VMEM),
        ],
        out_specs=pl.BlockSpec(memory_space=pl.ANY),
        out_shape=jax.ShapeDtypeStruct((_N, _H), jnp.float32),
        scratch_shapes=[
            [pltpu.VMEM((_CHR, _H), jnp.float32) for _ in range(_NBUF)],
            pltpu.SemaphoreType.DMA((_NBUF,)),
            pltpu.SemaphoreType.DMA((_NBUF,)),
        ],
    )(alpha1, h, m, scale2, shift2)
    return out.reshape(_B, _S, _H)
